# baseline (speedup 1.0000x reference)
import functools

import jax
import jax.numpy as jnp
from jax import lax
from jax.experimental import pallas as pl
from jax.experimental.pallas import tpu as pltpu
from jax.experimental.pallas import tpu_sc as plsc

BS = 1024
D = 64
NN = 501
NPAD = 512
NC = 2
NS = 16
NW = NC * NS
BPW = BS // NW
NCHUNK = 128
L = 16


def _sincos_body(day_ref, w1_ref, w2_ref, dreal_ref, dimg_ref):
    ang1 = w1_ref[:] * day_ref[:]
    ang2 = w2_ref[:] * day_ref[:]
    dimg_ref[:] = jnp.sin(ang1)
    dreal_ref[:] = jnp.cos(ang2)


def _sincos(day, w1, w2):
    return pl.pallas_call(
        _sincos_body,
        out_shape=(
            jax.ShapeDtypeStruct((BS, D), jnp.float32),
            jax.ShapeDtypeStruct((BS, D), jnp.float32),
        ),
    )(day.reshape(BS, 1), w1.reshape(1, D), w2.reshape(1, D))


def _ce_body(scores_ref, out_ref):
    s = scores_ref[:]
    col = lax.broadcasted_iota(jnp.int32, (BS, NPAD), 1)
    s = jnp.where(col < NN, s, -1e30)
    m = jnp.max(s, axis=1, keepdims=True)
    lse = m[:, 0] + jnp.log(jnp.sum(jnp.exp(s - m), axis=1))
    loss = lse - s[:, 0]
    out_ref[0, 0] = jnp.sum(loss) * (1.0 / BS)


def _ce(scores):
    return pl.pallas_call(
        _ce_body,
        out_shape=jax.ShapeDtypeStruct((1, 1), jnp.float32),
        out_specs=pl.BlockSpec(memory_space=pltpu.SMEM),
    )(scores)


def _sc_scores_body(sub_hbm, rel_hbm, ent_hbm, dr_hbm, di_hbm,
                    eR_hbm, eI_hbm, rR_hbm, rI_hbm, out_hbm,
                    sub_v, rel_v, esr_v, esi_v, rr_v, ri_v, dc_v, ds_v,
                    idx_v, er_v, ei_v, scores_v, sem1, sem2):
    wid = lax.axis_index("s") * NC + lax.axis_index("c")
    base = wid * BPW

    pltpu.sync_copy(sub_hbm.at[pl.ds(base, BPW)], sub_v)
    pltpu.sync_copy(rel_hbm.at[pl.ds(base, BPW)], rel_v)
    pltpu.sync_copy(dr_hbm.at[pl.ds(base, BPW)], dc_v)
    pltpu.sync_copy(di_hbm.at[pl.ds(base, BPW)], ds_v)
    c1 = pltpu.async_copy(eR_hbm.at[sub_v], esr_v, sem1)
    c2 = pltpu.async_copy(eI_hbm.at[sub_v], esi_v, sem2)
    c1.wait()
    c2.wait()
    c1 = pltpu.async_copy(rR_hbm.at[rel_v], rr_v, sem1)
    c2 = pltpu.async_copy(rI_hbm.at[rel_v], ri_v, sem2)
    c1.wait()
    c2.wait()

    lane = lax.iota(jnp.int32, L)
    perms = [jnp.bitwise_xor(lane, k) for k in (8, 4, 2, 1)]

    dnums = lax.GatherDimensionNumbers(
        offset_dims=(), collapsed_slice_dims=(0,), start_index_map=(0,))

    def _allreduce_sum(v):
        for p in perms:
            shuf = lax.gather(v, p[:, None], dnums, (1,),
                              mode=lax.GatherScatterMode.PROMISE_IN_BOUNDS)
            v = v + shuf
        return v

    def bi_body(bi, _):
        b = base + bi
        arc = []
        aic = []
        dcc = []
        dsc = []
        for c in range(4):
            sl = pl.ds(c * L, L)
            esr = esr_v[bi, sl]
            esi = esi_v[bi, sl]
            dc = dc_v[bi, sl]
            dsn = ds_v[bi, sl]
            arc.append(esr * dc - esi * dsn + rr_v[bi, sl])
            aic.append(esr * dsn + esi * dc + ri_v[bi, sl])
            dcc.append(dc)
            dsc.append(dsn)

        for nc in range(NPAD // NCHUNK):
            off = b * NPAD + nc * NCHUNK
            pltpu.sync_copy(ent_hbm.at[pl.ds(off, NCHUNK)], idx_v)
            g1 = pltpu.async_copy(eR_hbm.at[idx_v], er_v, sem1)
            g2 = pltpu.async_copy(eI_hbm.at[idx_v], ei_v, sem2)
            g1.wait()
            g2.wait()

            def g_body(g, _g):
                def pair_body(j, svec):
                    n = g * L + j
                    acc = jnp.zeros((L,), jnp.float32)
                    for c in range(4):
                        sl = pl.ds(c * L, L)
                        er = er_v[n, sl]
                        ei = ei_v[n, sl]
                        tr = er * dcc[c] - ei * dsc[c]
                        ti = er * dsc[c] + ei * dcc[c]
                        acc = acc + jnp.abs(arc[c] - tr) + jnp.abs(aic[c] + ti)
                    s = _allreduce_sum(acc)
                    return jnp.where(lane == j, s, svec)

                svec = lax.fori_loop(0, L, pair_body, jnp.zeros((L,), jnp.float32))
                scores_v[pl.ds(nc * NCHUNK + g * L, L)] = svec
                return 0

            lax.fori_loop(0, NCHUNK // L, g_body, 0)

        pltpu.sync_copy(scores_v, out_hbm.at[b])
        return 0

    lax.fori_loop(0, BPW, bi_body, 0)


@functools.cache
def _build_sc_scores():
    return functools.partial(
        pl.kernel,
        mesh=plsc.VectorSubcoreMesh(core_axis_name="c", subcore_axis_name="s"),
        out_type=jax.ShapeDtypeStruct((BS, NPAD), jnp.float32),
        compiler_params=pltpu.CompilerParams(use_tc_tiling_on_sc=False),
        scratch_types=[
            pltpu.VMEM((BPW,), jnp.int32),
            pltpu.VMEM((BPW,), jnp.int32),
            pltpu.VMEM((BPW, D), jnp.float32),
            pltpu.VMEM((BPW, D), jnp.float32),
            pltpu.VMEM((BPW, D), jnp.float32),
            pltpu.VMEM((BPW, D), jnp.float32),
            pltpu.VMEM((BPW, D), jnp.float32),
            pltpu.VMEM((BPW, D), jnp.float32),
            pltpu.VMEM((NCHUNK,), jnp.int32),
            pltpu.VMEM((NCHUNK, D), jnp.float32),
            pltpu.VMEM((NCHUNK, D), jnp.float32),
            pltpu.VMEM((NPAD,), jnp.float32),
            pltpu.SemaphoreType.DMA,
            pltpu.SemaphoreType.DMA,
        ],
    )(_sc_scores_body)


def kernel(sub, rel, obj, year, month, day, neg, emb_E_real, emb_E_img,
           emb_R_real, emb_R_img, w1, w2):
    del year, month
    ent = jnp.concatenate([obj[:, None], neg], axis=1).astype(jnp.int32)
    ent = jnp.pad(ent, ((0, 0), (0, NPAD - NN)))
    ent_flat = ent.reshape(-1)
    d_real, d_img = _sincos(day, w1, w2)
    scores = _build_sc_scores()(sub.astype(jnp.int32), rel.astype(jnp.int32),
                                ent_flat, d_real, d_img,
                                emb_E_real, emb_E_img, emb_R_real, emb_R_img)
    return _ce(scores)[0, 0]


# speedup vs baseline: 1.4859x; 1.4859x over previous
import functools

import jax
import jax.numpy as jnp
from jax import lax
from jax.experimental import pallas as pl
from jax.experimental.pallas import tpu as pltpu
from jax.experimental.pallas import tpu_sc as plsc

BS = 1024
D = 64
NN = 501
NPAD = 512
NC = 2
NS = 16
NW = NC * NS
BPW = BS // NW
NCHUNK = 128
L = 16


def _sincos_body(day_ref, w1_ref, w2_ref, dreal_ref, dimg_ref):
    ang1 = w1_ref[:] * day_ref[:]
    ang2 = w2_ref[:] * day_ref[:]
    dimg_ref[:] = jnp.sin(ang1)
    dreal_ref[:] = jnp.cos(ang2)


def _sincos(day, w1, w2):
    return pl.pallas_call(
        _sincos_body,
        out_shape=(
            jax.ShapeDtypeStruct((BS, D), jnp.float32),
            jax.ShapeDtypeStruct((BS, D), jnp.float32),
        ),
    )(day.reshape(BS, 1), w1.reshape(1, D), w2.reshape(1, D))


def _pack_body(tr_ref, ti_ref, out_ref):
    out_ref[:, 0:D] = tr_ref[:].T
    out_ref[:, D:2 * D] = ti_ref[:].T


def _pack(eR, eI, n_rows, bw):
    return pl.pallas_call(
        _pack_body,
        grid=(pl.cdiv(n_rows, bw),),
        in_specs=[
            pl.BlockSpec((D, bw), lambda i: (0, i)),
            pl.BlockSpec((D, bw), lambda i: (0, i)),
        ],
        out_specs=pl.BlockSpec((bw, 2 * D), lambda i: (i, 0)),
        out_shape=jax.ShapeDtypeStruct((n_rows, 2 * D), jnp.float32),
    )(eR.T, eI.T)


def _ce_body(scores_ref, out_ref):
    s = scores_ref[:]
    col = lax.broadcasted_iota(jnp.int32, (BS, NPAD), 1)
    s = jnp.where(col < NN, s, -1e30)
    m = jnp.max(s, axis=1, keepdims=True)
    lse = m[:, 0] + jnp.log(jnp.sum(jnp.exp(s - m), axis=1))
    loss = lse - s[:, 0]
    out_ref[0, 0] = jnp.sum(loss) * (1.0 / BS)


def _ce(scores):
    return pl.pallas_call(
        _ce_body,
        out_shape=jax.ShapeDtypeStruct((1, 1), jnp.float32),
        out_specs=pl.BlockSpec(memory_space=pltpu.SMEM),
    )(scores)


def _sc_scores_body(sub_hbm, rel_hbm, ent_hbm, dr_hbm, di_hbm,
                    tabE_hbm, tabR_hbm, out_hbm,
                    sub_v, rel_v, esub_v, rrow_v, dc_v, ds_v,
                    idx_v, rows_v, scores_v, sem1, sem2):
    wid = lax.axis_index("s") * NC + lax.axis_index("c")
    base = wid * BPW

    pltpu.sync_copy(sub_hbm.at[pl.ds(base, BPW)], sub_v)
    pltpu.sync_copy(rel_hbm.at[pl.ds(base, BPW)], rel_v)
    pltpu.sync_copy(dr_hbm.at[pl.ds(base, BPW)], dc_v)
    pltpu.sync_copy(di_hbm.at[pl.ds(base, BPW)], ds_v)
    c1 = pltpu.async_copy(tabE_hbm.at[sub_v], esub_v, sem1)
    c2 = pltpu.async_copy(tabR_hbm.at[rel_v], rrow_v, sem2)
    c1.wait()
    c2.wait()

    lane = lax.iota(jnp.int32, L)
    perms = [jnp.bitwise_xor(lane, k) for k in (8, 4, 2, 1)]

    dnums = lax.GatherDimensionNumbers(
        offset_dims=(), collapsed_slice_dims=(0,), start_index_map=(0,))

    def _allreduce_sum(v):
        for p in perms:
            shuf = lax.gather(v, p[:, None], dnums, (1,),
                              mode=lax.GatherScatterMode.PROMISE_IN_BOUNDS)
            v = v + shuf
        return v

    def bi_body(bi, _):
        b = base + bi
        arc = []
        aic = []
        dcc = []
        dsc = []
        for c in range(4):
            sl = pl.ds(c * L, L)
            isl = pl.ds(D + c * L, L)
            esr = esub_v[bi, sl]
            esi = esub_v[bi, isl]
            dc = dc_v[bi, sl]
            dsn = ds_v[bi, sl]
            arc.append(esr * dc - esi * dsn + rrow_v[bi, sl])
            aic.append(esr * dsn + esi * dc + rrow_v[bi, isl])
            dcc.append(dc)
            dsc.append(dsn)

        for nc in range(NPAD // NCHUNK):
            off = b * NPAD + nc * NCHUNK
            pltpu.sync_copy(ent_hbm.at[pl.ds(off, NCHUNK)], idx_v)
            pltpu.async_copy(tabE_hbm.at[idx_v], rows_v, sem1).wait()

            def g_body(g, _g):
                def pair_body(j, svec):
                    n = g * L + j
                    acc = jnp.zeros((L,), jnp.float32)
                    for c in range(4):
                        sl = pl.ds(c * L, L)
                        er = rows_v[n, sl]
                        ei = rows_v[n, pl.ds(D + c * L, L)]
                        tr = er * dcc[c] - ei * dsc[c]
                        ti = er * dsc[c] + ei * dcc[c]
                        acc = acc + jnp.abs(arc[c] - tr) + jnp.abs(aic[c] + ti)
                    s = _allreduce_sum(acc)
                    return jnp.where(lane == j, s, svec)

                svec = lax.fori_loop(0, L, pair_body, jnp.zeros((L,), jnp.float32))
                scores_v[pl.ds(nc * NCHUNK + g * L, L)] = svec
                return 0

            lax.fori_loop(0, NCHUNK // L, g_body, 0)

        pltpu.sync_copy(scores_v, out_hbm.at[b])
        return 0

    lax.fori_loop(0, BPW, bi_body, 0)


@functools.cache
def _build_sc_scores():
    return functools.partial(
        pl.kernel,
        mesh=plsc.VectorSubcoreMesh(core_axis_name="c", subcore_axis_name="s"),
        out_type=jax.ShapeDtypeStruct((BS, NPAD), jnp.float32),
        compiler_params=pltpu.CompilerParams(use_tc_tiling_on_sc=False),
        scratch_types=[
            pltpu.VMEM((BPW,), jnp.int32),
            pltpu.VMEM((BPW,), jnp.int32),
            pltpu.VMEM((BPW, 2 * D), jnp.float32),
            pltpu.VMEM((BPW, 2 * D), jnp.float32),
            pltpu.VMEM((BPW, D), jnp.float32),
            pltpu.VMEM((BPW, D), jnp.float32),
            pltpu.VMEM((NCHUNK,), jnp.int32),
            pltpu.VMEM((NCHUNK, 2 * D), jnp.float32),
            pltpu.VMEM((NPAD,), jnp.float32),
            pltpu.SemaphoreType.DMA,
            pltpu.SemaphoreType.DMA,
        ],
    )(_sc_scores_body)


def kernel(sub, rel, obj, year, month, day, neg, emb_E_real, emb_E_img,
           emb_R_real, emb_R_img, w1, w2):
    del year, month
    ent = jnp.concatenate([obj[:, None], neg], axis=1).astype(jnp.int32)
    ent = jnp.pad(ent, ((0, 0), (0, NPAD - NN)))
    ent_flat = ent.reshape(-1)
    d_real, d_img = _sincos(day, w1, w2)
    tabE = _pack(emb_E_real, emb_E_img, 1000000, 8192)
    tabR = _pack(emb_R_real, emb_R_img, 1000, 1000)
    scores = _build_sc_scores()(sub.astype(jnp.int32), rel.astype(jnp.int32),
                                ent_flat, d_real, d_img, tabE, tabR)
    return _ce(scores)[0, 0]


# speedup vs baseline: 1.5025x; 1.0112x over previous
import functools

import jax
import jax.numpy as jnp
from jax import lax
from jax.experimental import pallas as pl
from jax.experimental.pallas import tpu as pltpu
from jax.experimental.pallas import tpu_sc as plsc

BS = 1024
D = 64
NN = 501
NPAD = 512
NC = 2
NS = 16
NW = NC * NS
BPW = BS // NW
NCHUNK = 128
L = 16


def _sincos_body(day_ref, w1_ref, w2_ref, dreal_ref, dimg_ref):
    ang1 = w1_ref[:] * day_ref[:]
    ang2 = w2_ref[:] * day_ref[:]
    dimg_ref[:] = jnp.sin(ang1)
    dreal_ref[:] = jnp.cos(ang2)


def _sincos(day, w1, w2):
    return pl.pallas_call(
        _sincos_body,
        out_shape=(
            jax.ShapeDtypeStruct((BS, D), jnp.float32),
            jax.ShapeDtypeStruct((BS, D), jnp.float32),
        ),
    )(day.reshape(BS, 1), w1.reshape(1, D), w2.reshape(1, D))


def _pack_body(tr_ref, ti_ref, out_ref):
    out_ref[:, 0:D] = tr_ref[:].T
    out_ref[:, D:2 * D] = ti_ref[:].T


def _pack(eR, eI, n_rows, bw):
    return pl.pallas_call(
        _pack_body,
        grid=(pl.cdiv(n_rows, bw),),
        in_specs=[
            pl.BlockSpec((D, bw), lambda i: (0, i)),
            pl.BlockSpec((D, bw), lambda i: (0, i)),
        ],
        out_specs=pl.BlockSpec((bw, 2 * D), lambda i: (i, 0)),
        out_shape=jax.ShapeDtypeStruct((n_rows, 2 * D), jnp.float32),
    )(eR.T, eI.T)


def _ce_body(scores_ref, out_ref):
    s = scores_ref[:]
    col = lax.broadcasted_iota(jnp.int32, (BS, NPAD), 1)
    s = jnp.where(col < NN, s, -1e30)
    m = jnp.max(s, axis=1, keepdims=True)
    lse = m[:, 0] + jnp.log(jnp.sum(jnp.exp(s - m), axis=1))
    loss = lse - s[:, 0]
    out_ref[0, 0] = jnp.sum(loss) * (1.0 / BS)


def _ce(scores):
    return pl.pallas_call(
        _ce_body,
        out_shape=jax.ShapeDtypeStruct((1, 1), jnp.float32),
        out_specs=pl.BlockSpec(memory_space=pltpu.SMEM),
    )(scores)


def _sc_scores_body(sub_hbm, rel_hbm, ent_hbm, dr_hbm, di_hbm,
                    tabE_hbm, tabR_hbm, out_hbm,
                    sub_v, rel_v, esub_v, rrow_v, dc_v, ds_v,
                    idxall_v, rows0_v, rows1_v, scores_v, sem0, sem1):
    wid = lax.axis_index("s") * NC + lax.axis_index("c")
    base = wid * BPW

    pltpu.sync_copy(sub_hbm.at[pl.ds(base, BPW)], sub_v)
    pltpu.sync_copy(rel_hbm.at[pl.ds(base, BPW)], rel_v)
    pltpu.sync_copy(dr_hbm.at[pl.ds(base, BPW)], dc_v)
    pltpu.sync_copy(di_hbm.at[pl.ds(base, BPW)], ds_v)
    pltpu.sync_copy(ent_hbm.at[pl.ds(base * NPAD, BPW * NPAD)], idxall_v)
    c1 = pltpu.async_copy(tabE_hbm.at[sub_v], esub_v, sem0)
    c2 = pltpu.async_copy(tabR_hbm.at[rel_v], rrow_v, sem1)
    c1.wait()
    c2.wait()

    lane = lax.iota(jnp.int32, L)
    perms = [jnp.bitwise_xor(lane, k) for k in (8, 4, 2, 1)]

    dnums = lax.GatherDimensionNumbers(
        offset_dims=(), collapsed_slice_dims=(0,), start_index_map=(0,))

    def _allreduce_sum(v):
        for p in perms:
            shuf = lax.gather(v, p[:, None], dnums, (1,),
                              mode=lax.GatherScatterMode.PROMISE_IN_BOUNDS)
            v = v + shuf
        return v

    rows = (rows0_v, rows1_v)
    sems = (sem0, sem1)
    K = BPW * (NPAD // NCHUNK)

    def _issue(j, par):
        pltpu.async_copy(
            tabE_hbm.at[idxall_v.at[pl.ds(j * NCHUNK, NCHUNK)]],
            rows[par], sems[par])

    _issue(0, 0)
    _issue(1, 1)

    def g2_body(g2, _):
        for par in range(2):
            k = 2 * g2 + par
            bi = k // (NPAD // NCHUNK)
            nc = k % (NPAD // NCHUNK)
            rv = rows[par]
            pltpu.make_async_copy(
                tabE_hbm.at[idxall_v.at[pl.ds(0, NCHUNK)]], rv, sems[par]
            ).wait()

            arc = []
            aic = []
            dcc = []
            dsc = []
            for c in range(4):
                sl = pl.ds(c * L, L)
                isl = pl.ds(D + c * L, L)
                esr = esub_v[bi, sl]
                esi = esub_v[bi, isl]
                dc = dc_v[bi, sl]
                dsn = ds_v[bi, sl]
                arc.append(esr * dc - esi * dsn + rrow_v[bi, sl])
                aic.append(esr * dsn + esi * dc + rrow_v[bi, isl])
                dcc.append(dc)
                dsc.append(dsn)

            def g_body(g, _g):
                def pair_body(j, svec):
                    n = g * L + j
                    acc = jnp.zeros((L,), jnp.float32)
                    for c in range(4):
                        sl = pl.ds(c * L, L)
                        er = rv[n, sl]
                        ei = rv[n, pl.ds(D + c * L, L)]
                        tr = er * dcc[c] - ei * dsc[c]
                        ti = er * dsc[c] + ei * dcc[c]
                        acc = acc + jnp.abs(arc[c] - tr) + jnp.abs(aic[c] + ti)
                    s = _allreduce_sum(acc)
                    return jnp.where(lane == j, s, svec)

                svec = lax.fori_loop(0, L, pair_body,
                                     jnp.zeros((L,), jnp.float32))
                scores_v[pl.ds(nc * NCHUNK + g * L, L)] = svec
                return 0

            lax.fori_loop(0, NCHUNK // L, g_body, 0)

            @pl.when(k + 2 < K)
            def _():
                _issue(k + 2, par)

            @pl.when(nc == NPAD // NCHUNK - 1)
            def _():
                pltpu.sync_copy(scores_v, out_hbm.at[base + bi])
        return 0

    lax.fori_loop(0, K // 2, g2_body, 0)


@functools.cache
def _build_sc_scores():
    return functools.partial(
        pl.kernel,
        mesh=plsc.VectorSubcoreMesh(core_axis_name="c", subcore_axis_name="s"),
        out_type=jax.ShapeDtypeStruct((BS, NPAD), jnp.float32),
        compiler_params=pltpu.CompilerParams(use_tc_tiling_on_sc=False),
        scratch_types=[
            pltpu.VMEM((BPW,), jnp.int32),
            pltpu.VMEM((BPW,), jnp.int32),
            pltpu.VMEM((BPW, 2 * D), jnp.float32),
            pltpu.VMEM((BPW, 2 * D), jnp.float32),
            pltpu.VMEM((BPW, D), jnp.float32),
            pltpu.VMEM((BPW, D), jnp.float32),
            pltpu.VMEM((BPW * NPAD,), jnp.int32),
            pltpu.VMEM((NCHUNK, 2 * D), jnp.float32),
            pltpu.VMEM((NCHUNK, 2 * D), jnp.float32),
            pltpu.VMEM((NPAD,), jnp.float32),
            pltpu.SemaphoreType.DMA,
            pltpu.SemaphoreType.DMA,
        ],
    )(_sc_scores_body)


def kernel(sub, rel, obj, year, month, day, neg, emb_E_real, emb_E_img,
           emb_R_real, emb_R_img, w1, w2):
    del year, month
    ent = jnp.concatenate([obj[:, None], neg], axis=1).astype(jnp.int32)
    ent = jnp.pad(ent, ((0, 0), (0, NPAD - NN)))
    ent_flat = ent.reshape(-1)
    d_real, d_img = _sincos(day, w1, w2)
    tabE = _pack(emb_E_real, emb_E_img, 1000000, 8192)
    tabR = _pack(emb_R_real, emb_R_img, 1000, 1000)
    scores = _build_sc_scores()(sub.astype(jnp.int32), rel.astype(jnp.int32),
                                ent_flat, d_real, d_img, tabE, tabR)
    return _ce(scores)[0, 0]
